# baseline (speedup 1.0000x reference)
import functools

import jax
import jax.numpy as jnp
from jax import lax
from jax.experimental import pallas as pl
from jax.experimental.pallas import tpu as pltpu
from jax.experimental.pallas import tpu_sc as plsc

B, E, N, D = 16, 32000, 1000, 64
NP = 1024
NH = NP // 2
L = 16
CH = 6400
NCH = E // CH
MCAP = 2048

_sc_mesh = plsc.VectorSubcoreMesh(core_axis_name="c", subcore_axis_name="s")


def _sc_body(ei_hbm, act_hbm, ea_hbm, gath_out, win_out,
             row_v, col_v, win_v, mev_v, mcol_v, tbuf, rows_v, act_v, sem):
    c = lax.axis_index("c")
    s = lax.axis_index("s")

    @pl.when(s < 8)
    def _():
        b = c * 8 + s
        lanes = lax.iota(jnp.int32, L)
        pltpu.sync_copy(act_hbm, act_v)
        a = plsc.load_gather(act_v, [jnp.full((L,), b, jnp.int32)])

        def init_body(i, _):
            win_v[pl.ds(i * L, L)] = jnp.full((L,), -1, jnp.int32)
            return 0
        lax.fori_loop(0, NP // L, init_body, 0)

        cnt = jnp.int32(0)
        for g in range(NCH):
            pltpu.sync_copy(ei_hbm.at[pl.ds(b * 2 * E + g * CH, CH)], row_v)
            pltpu.sync_copy(ei_hbm.at[pl.ds(b * 2 * E + E + g * CH, CH)], col_v)

            def scan_body(i, cnt, g=g):
                r = row_v[pl.ds(i * L, L)]
                cidx = col_v[pl.ds(i * L, L)]
                ev = jnp.int32(g * CH) + i * L + lanes
                m = r == a
                plsc.store_scatter(win_v, [cidx], ev, mask=m)
                cl = jnp.minimum(cnt, MCAP)
                plsc.store_compressed(mev_v.at[pl.ds(cl, L)], ev, mask=m)
                plsc.store_compressed(mcol_v.at[pl.ds(cl, L)], cidx, mask=m)
                return cnt + plsc.all_reduce_population_count(m)[0]
            cnt = lax.fori_loop(0, CH // L, scan_body, cnt)

        def fetch_body(i, _):
            w = mev_v[pl.ds(i, L)][0]
            cc = mcol_v[pl.ds(i, L)][0]
            w128 = pl.multiple_of((w // 128) * 128, 128)
            pltpu.sync_copy(ea_hbm.at[b, :, pl.ds(w128, 128)], tbuf)
            wsub = jnp.full((L,), w - w128, jnp.int32)
            half = (cc % 2) * D
            for k in range(D // L):
                rows_v[cc // 2, pl.ds(half + k * L, L)] = \
                    plsc.load_gather(tbuf, [lanes + k * L, wsub])
            return 0
        lax.fori_loop(0, jnp.minimum(cnt, MCAP), fetch_body, 0)

        pltpu.sync_copy(rows_v, gath_out.at[pl.ds(b * NH, NH)])
        pltpu.sync_copy(win_v, win_out.at[pl.ds(b * NP, NP)])


_sc_select = pl.kernel(
    _sc_body,
    out_type=(
        jax.ShapeDtypeStruct((B * NH, 2 * D), jnp.float32),
        jax.ShapeDtypeStruct((B * NP,), jnp.int32),
    ),
    mesh=_sc_mesh,
    compiler_params=pltpu.CompilerParams(needs_layout_passes=False),
    scratch_types=[
        pltpu.VMEM((CH,), jnp.int32),
        pltpu.VMEM((CH,), jnp.int32),
        pltpu.VMEM((NP,), jnp.int32),
        pltpu.VMEM((MCAP + L,), jnp.int32),
        pltpu.VMEM((MCAP + L,), jnp.int32),
        pltpu.VMEM((D, 128), jnp.float32),
        pltpu.VMEM((NH, 2 * D), jnp.float32),
        pltpu.VMEM((L,), jnp.int32),
        pltpu.SemaphoreType.DMA,
    ],
)


def _mlp_body(g_ref, wn_ref, am_ref, w0_ref, b0_ref, w1_ref, b1_ref,
              w2_ref, b2_ref, wo_ref, bo_ref, lp_ref, mk_ref):
    g = g_ref[...]
    x = jnp.concatenate([g[:, :D], g[:, D:]], axis=0)
    for w_r, b_r in ((w0_ref, b0_ref), (w1_ref, b1_ref), (w2_ref, b2_ref)):
        y = lax.dot_general(x, w_r[...], (((1,), (1,)), ((), ())),
                            preferred_element_type=jnp.float32)
        y = y + b_r[...][None, :]
        x = y * jax.nn.sigmoid(y)
    logits = jnp.sum(x * wo_ref[...], axis=1) + bo_ref[0]
    wn = wn_ref[...]
    am = am_ref[...]
    lp = jnp.where(wn >= 0, logits, jnp.float32(-1e9))
    lp_ref[...] = jnp.where(am == 0, jnp.float32(-jnp.inf), lp)
    mk_ref[...] = (am == 0).astype(jnp.int8)


_mlp_call = pl.pallas_call(
    _mlp_body,
    out_shape=[
        jax.ShapeDtypeStruct((B * NP,), jnp.float32),
        jax.ShapeDtypeStruct((B * NP,), jnp.int8),
    ],
)


def kernel(edge_attr, edge_index, action, action_mask,
           W0, b0, W1, b1, W2, b2, Wout, bout):
    ei = edge_index.astype(jnp.int32).reshape(-1)
    act = action.astype(jnp.int32)
    gath, win = _sc_select(ei, act, edge_attr.transpose(0, 2, 1))
    am_pad = jnp.pad(action_mask, ((0, 0), (0, NP - N))).astype(jnp.int32)
    win2 = win.reshape(B * NH, 2)
    am2 = am_pad.reshape(B * NH, 2)
    wn_cat = jnp.concatenate([win2[:, 0], win2[:, 1]])
    am_cat = jnp.concatenate([am2[:, 0], am2[:, 1]])
    lp_flat, mk_flat = _mlp_call(
        gath, wn_cat, am_cat, W0, b0, W1, b1, W2, b2, Wout, bout)
    lp2 = jnp.stack([lp_flat[:B * NH].reshape(B, NH),
                     lp_flat[B * NH:].reshape(B, NH)], axis=-1)
    mk2 = jnp.stack([mk_flat[:B * NH].reshape(B, NH),
                     mk_flat[B * NH:].reshape(B, NH)], axis=-1)
    log_p = lp2.reshape(B, NP)[:, :N]
    mask = mk2.reshape(B, NP)[:, :N].astype(bool)
    return log_p, mask
